# baseline (speedup 1.0000x reference)
import functools
import math

import jax
import jax.numpy as jnp
from jax import lax
from jax.experimental import pallas as pl
from jax.experimental.pallas import tpu as pltpu
from jax.experimental.pallas import tpu_sc as plsc

_SAMPLES = 9
_NLVL = 4
_G = 4
_GS = _G * _SAMPLES
_CH = _NLVL * _GS
_LANES = 16
_HB = 8


def _pool_matrix(wl, wn, scale):
    r = lax.broadcasted_iota(jnp.int32, (wl, wn), 0)
    c = lax.broadcasted_iota(jnp.int32, (wl, wn), 1)
    return jnp.where(r // 2 == c, jnp.float32(scale), jnp.float32(0.0))


def _corr_pyr_body(f1_ref, f2_ref, o0_ref, o1_ref, o23_ref):
    c = f1_ref.shape[1]
    scale = jnp.float32(1.0 / math.sqrt(float(c)))
    p1 = _pool_matrix(256, 128, 0.5)
    r = lax.broadcasted_iota(jnp.int32, (128, 128), 0)
    cc = lax.broadcasted_iota(jnp.int32, (128, 128), 1)
    q = jnp.where(
        (cc < 64) & (r // 2 == cc), jnp.float32(0.5),
        jnp.where((cc >= 64) & (cc < 96) & (r // 4 == cc - 64),
                  jnp.float32(0.25), jnp.float32(0.0)))
    for hh in range(_HB):
        f1 = f1_ref[0, :, hh, :]
        f2 = f2_ref[0, :, hh, :]
        corr = lax.dot_general(f1, f2, (((0,), (0,)), ((), ())),
                               preferred_element_type=jnp.float32) * scale
        o0_ref[0, hh] = jnp.concatenate([corr[:, :128], corr[:, 128:]], axis=0)
        l1 = lax.dot_general(corr, p1, (((1,), (0,)), ((), ())),
                             preferred_element_type=jnp.float32)
        o1_ref[0, hh] = l1
        o23_ref[0, hh] = lax.dot_general(l1, q, (((1,), (0,)), ((), ())),
                                         preferred_element_type=jnp.float32)


def _corr_pyramid(f1, f2):
    b, c, h, w = f1.shape
    shapes = [(b, h, 2 * w, 128), (b, h, w, 128), (b, h, w, 128)]
    return pl.pallas_call(
        _corr_pyr_body,
        grid=(b, h // _HB),
        in_specs=[pl.BlockSpec((1, c, _HB, w), lambda i, j: (i, 0, j, 0))] * 2,
        out_specs=[pl.BlockSpec((1, _HB, s[2], 128), lambda i, j: (i, j, 0, 0))
                   for s in shapes],
        out_shape=[jax.ShapeDtypeStruct(s, jnp.float32) for s in shapes],
    )(f1, f2)


def _make_sc_sampler(b_sz, h_sz, w):
    level_w = [w // (2 ** l) for l in range(_NLVL)]
    n_chunks = b_sz * h_sz * (w // _LANES)
    mesh = plsc.VectorSubcoreMesh(core_axis_name="c", subcore_axis_name="s")
    wc = w // _LANES
    hcw = _CH // 2

    @functools.partial(
        pl.kernel,
        mesh=mesh,
        compiler_params=pltpu.CompilerParams(needs_layout_passes=False),
        out_type=jax.ShapeDtypeStruct((b_sz, _CH, h_sz, w), jnp.float32),
        scratch_types=(
            [pltpu.VMEM((_LANES, 256), jnp.float32) for _ in range(2)]
            + [pltpu.VMEM((_LANES, 128), jnp.float32) for _ in range(4)]
            + [pltpu.VMEM((_G, 8, 128), jnp.float32) for _ in range(2)]
            + [pltpu.VMEM((hcw, 8, 128), jnp.float32)]
            + [pltpu.SemaphoreType.DMA for _ in range(3)]
        ),
    )
    def sampler(p0_hbm, p1_hbm, p23_hbm, crd_hbm, sig_hbm, out_hbm,
                p0a, p0b, p1a, p1b, p23a, p23b,
                c_v, s_v, o_v, sem_a, sem_b, sem_o):
        wid = lax.axis_index("s") * 2 + lax.axis_index("c")
        bi = lax.shift_right_logical(wid, 4)
        hb = lax.shift_right_logical(wid, 1) & 7
        wq = wid & 1
        lane = lax.broadcasted_iota(jnp.int32, (_LANES,), 0)
        bufs = [[p0a, p1a, p23a, p23a], [p0b, p1b, p23b, p23b]]
        sem_in = [sem_a, sem_b]

        def chunk_coords(k):
            hi = hb * 8 + lax.shift_right_logical(k, 3)
            s = wq * 8 + (k & 7)
            t = (bi * h_sz + hi) * wc + s
            return hi, s, t

        def in_copies(k, p, half):
            hi, s, t = chunk_coords(k)
            r0 = s * _LANES
            if half == 0:
                p0_v = bufs[p][0]
                yield pltpu.make_async_copy(
                    p0_hbm.at[bi, hi, pl.ds(r0, _LANES)],
                    p0_v.at[:, pl.ds(0, 128)], sem_in[p])
                yield pltpu.make_async_copy(
                    p0_hbm.at[bi, hi, pl.ds(w + r0, _LANES)],
                    p0_v.at[:, pl.ds(128, 128)], sem_in[p])
                yield pltpu.make_async_copy(
                    p1_hbm.at[bi, hi, pl.ds(r0, _LANES)], bufs[p][1],
                    sem_in[p])
            else:
                yield pltpu.make_async_copy(
                    p23_hbm.at[bi, hi, pl.ds(r0, _LANES)], bufs[p][2],
                    sem_in[p])

        def coord_copies(sem):
            yield pltpu.make_async_copy(
                crd_hbm.at[bi, :, pl.ds(hb * 8, 8), pl.ds(wq * 128, 128)],
                c_v, sem)
            yield pltpu.make_async_copy(
                sig_hbm.at[bi, :, pl.ds(hb * 8, 8), pl.ds(wq * 128, 128)],
                s_v, sem)

        def start_in(k, p, half):
            for cp in in_copies(k, p, half):
                cp.start()

        def wait_in(k, p, half):
            for cp in in_copies(k, p, half):
                cp.wait()

        def out_copy(half):
            return pltpu.make_async_copy(
                o_v,
                out_hbm.at[bi, pl.ds(half * hcw, hcw),
                           pl.ds(hb * 8, 8), pl.ds(wq * 128, 128)],
                sem_o)

        def compute(k, p, half):
            h_off = lax.shift_right_logical(k, 3)
            woff = (k & 7) * _LANES
            levels = (0, 1) if half == 0 else (2, 3)
            for g in range(_G):
                cg = c_v[g, h_off, pl.ds(woff, _LANES)]
                sg = s_v[g, h_off, pl.ds(woff, _LANES)]
                for s in range(_SAMPLES):
                    x = cg + jnp.float32(s - _SAMPLES // 2) * sg
                    xt = x.astype(jnp.int32)
                    f0 = jnp.where(x < xt.astype(jnp.float32), xt - 1, xt)
                    for l in levels:
                        wl = level_w[l]
                        xi = x * jnp.float32(1.0 / (2 ** l)) if l else x
                        f = lax.shift_right_arithmetic(f0, l) if l else f0
                        w1 = xi - f.astype(jnp.float32)
                        i1 = f + 1
                        c0 = jnp.clip(f, 0, wl - 1)
                        c1 = jnp.clip(i1, 0, wl - 1)
                        if l == 3:
                            v0 = plsc.load_gather(bufs[p][3], [lane, c0 + 64])
                            v1 = plsc.load_gather(bufs[p][3], [lane, c1 + 64])
                        else:
                            v0 = plsc.load_gather(bufs[p][l], [lane, c0])
                            v1 = plsc.load_gather(bufs[p][l], [lane, c1])
                        v0 = jnp.where(f == c0, v0, jnp.float32(0.0))
                        v1 = jnp.where(i1 == c1, v1, jnp.float32(0.0))
                        ch = l * _GS + g * _SAMPLES + s - half * hcw
                        o_v[ch, h_off, pl.ds(woff, _LANES)] = (
                            v0 + w1 * (v1 - v0))

        def run_pass(half, first):
            def body(k2, carry):
                for qp in range(2):
                    k = k2 * 2 + qp
                    wait_in(k, qp, half)
                    compute(k, qp, half)

                    @pl.when(k2 < 31)
                    def _start_next():
                        start_in(k + 2, qp, half)
                return carry

            if not first:
                out_copy(0).wait()
            lax.fori_loop(0, 32, body, 0)
            out_copy(half).start()

        for cp in coord_copies(sem_o):
            cp.start()
        start_in(0, 0, 0)
        start_in(1, 1, 0)
        for cp in coord_copies(sem_o):
            cp.wait()
        run_pass(0, True)
        start_in(0, 0, 1)
        start_in(1, 1, 1)
        run_pass(1, False)
        out_copy(1).wait()

    return sampler


def kernel(fmap1, fmap2, coords, sigma):
    b, c, h, w = fmap1.shape
    o0, o1, o23 = _corr_pyramid(fmap1, fmap2)
    sampler = _make_sc_sampler(b, h, w)
    return sampler(o0, o1, o23, coords, sigma)


# speedup vs baseline: 1.0589x; 1.0589x over previous
import functools
import math

import jax
import jax.numpy as jnp
from jax import lax
from jax.experimental import pallas as pl
from jax.experimental.pallas import tpu as pltpu
from jax.experimental.pallas import tpu_sc as plsc

_SAMPLES = 9
_NLVL = 4
_G = 4
_GS = _G * _SAMPLES
_CH = _NLVL * _GS
_LANES = 16
_HB = 16


def _pool_matrix(wl, wn, scale):
    r = lax.broadcasted_iota(jnp.int32, (wl, wn), 0)
    c = lax.broadcasted_iota(jnp.int32, (wl, wn), 1)
    return jnp.where(r // 2 == c, jnp.float32(scale), jnp.float32(0.0))


def _corr_pyr_body(f1_ref, f2_ref, o0_ref, o1_ref, o23_ref):
    c = f1_ref.shape[1]
    scale = jnp.float32(1.0 / math.sqrt(float(c)))
    p1 = _pool_matrix(256, 128, 0.5)
    r = lax.broadcasted_iota(jnp.int32, (128, 128), 0)
    cc = lax.broadcasted_iota(jnp.int32, (128, 128), 1)
    q = jnp.where(
        (cc < 64) & (r // 2 == cc), jnp.float32(0.5),
        jnp.where((cc >= 64) & (cc < 96) & (r // 4 == cc - 64),
                  jnp.float32(0.25), jnp.float32(0.0)))
    for hh in range(_HB):
        f1 = f1_ref[0, :, hh, :]
        f2 = f2_ref[0, :, hh, :]
        corr = lax.dot_general(f1, f2, (((0,), (0,)), ((), ())),
                               preferred_element_type=jnp.float32) * scale
        o0_ref[0, hh] = jnp.concatenate([corr[:, :128], corr[:, 128:]], axis=0)
        l1 = lax.dot_general(corr, p1, (((1,), (0,)), ((), ())),
                             preferred_element_type=jnp.float32)
        o1_ref[0, hh] = l1
        o23_ref[0, hh] = lax.dot_general(l1, q, (((1,), (0,)), ((), ())),
                                         preferred_element_type=jnp.float32)


def _corr_pyramid(f1, f2):
    b, c, h, w = f1.shape
    shapes = [(b, h, 2 * w, 128), (b, h, w, 128), (b, h, w, 128)]
    return pl.pallas_call(
        _corr_pyr_body,
        grid=(b, h // _HB),
        in_specs=[pl.BlockSpec((1, c, _HB, w), lambda i, j: (i, 0, j, 0))] * 2,
        out_specs=[pl.BlockSpec((1, _HB, s[2], 128), lambda i, j: (i, j, 0, 0))
                   for s in shapes],
        out_shape=[jax.ShapeDtypeStruct(s, jnp.float32) for s in shapes],
    )(f1, f2)


def _make_sc_sampler(b_sz, h_sz, w):
    level_w = [w // (2 ** l) for l in range(_NLVL)]
    n_chunks = b_sz * h_sz * (w // _LANES)
    mesh = plsc.VectorSubcoreMesh(core_axis_name="c", subcore_axis_name="s")
    wc = w // _LANES
    hcw = _CH // 2

    @functools.partial(
        pl.kernel,
        mesh=mesh,
        compiler_params=pltpu.CompilerParams(needs_layout_passes=False),
        out_type=jax.ShapeDtypeStruct((b_sz, _CH, h_sz, w), jnp.float32),
        scratch_types=(
            [pltpu.VMEM((_LANES, 256), jnp.float32) for _ in range(2)]
            + [pltpu.VMEM((_LANES, 128), jnp.float32) for _ in range(4)]
            + [pltpu.VMEM((_G, 8, 128), jnp.float32) for _ in range(2)]
            + [pltpu.VMEM((hcw, 8, 128), jnp.float32)]
            + [pltpu.SemaphoreType.DMA for _ in range(3)]
        ),
    )
    def sampler(p0_hbm, p1_hbm, p23_hbm, crd_hbm, sig_hbm, out_hbm,
                p0a, p0b, p1a, p1b, p23a, p23b,
                c_v, s_v, o_v, sem_a, sem_b, sem_o):
        wid = lax.axis_index("s") * 2 + lax.axis_index("c")
        bi = lax.shift_right_logical(wid, 4)
        hb = lax.shift_right_logical(wid, 1) & 7
        wq = wid & 1
        lane = lax.broadcasted_iota(jnp.int32, (_LANES,), 0)
        bufs = [[p0a, p1a, p23a, p23a], [p0b, p1b, p23b, p23b]]
        sem_in = [sem_a, sem_b]

        def chunk_coords(k):
            hi = hb * 8 + lax.shift_right_logical(k, 3)
            s = wq * 8 + (k & 7)
            t = (bi * h_sz + hi) * wc + s
            return hi, s, t

        def in_copies(k, p, half):
            hi, s, t = chunk_coords(k)
            r0 = s * _LANES
            if half == 0:
                p0_v = bufs[p][0]
                yield pltpu.make_async_copy(
                    p0_hbm.at[bi, hi, pl.ds(r0, _LANES)],
                    p0_v.at[:, pl.ds(0, 128)], sem_in[p])
                yield pltpu.make_async_copy(
                    p0_hbm.at[bi, hi, pl.ds(w + r0, _LANES)],
                    p0_v.at[:, pl.ds(128, 128)], sem_in[p])
                yield pltpu.make_async_copy(
                    p1_hbm.at[bi, hi, pl.ds(r0, _LANES)], bufs[p][1],
                    sem_in[p])
            else:
                yield pltpu.make_async_copy(
                    p23_hbm.at[bi, hi, pl.ds(r0, _LANES)], bufs[p][2],
                    sem_in[p])

        def coord_copies(sem):
            yield pltpu.make_async_copy(
                crd_hbm.at[bi, :, pl.ds(hb * 8, 8), pl.ds(wq * 128, 128)],
                c_v, sem)
            yield pltpu.make_async_copy(
                sig_hbm.at[bi, :, pl.ds(hb * 8, 8), pl.ds(wq * 128, 128)],
                s_v, sem)

        def start_in(k, p, half):
            for cp in in_copies(k, p, half):
                cp.start()

        def wait_in(k, p, half):
            for cp in in_copies(k, p, half):
                cp.wait()

        def out_copy(half):
            return pltpu.make_async_copy(
                o_v,
                out_hbm.at[bi, pl.ds(half * hcw, hcw),
                           pl.ds(hb * 8, 8), pl.ds(wq * 128, 128)],
                sem_o)

        def compute(k, p, half):
            h_off = lax.shift_right_logical(k, 3)
            woff = (k & 7) * _LANES
            levels = (0, 1) if half == 0 else (2, 3)
            for g in range(_G):
                cg = c_v[g, h_off, pl.ds(woff, _LANES)]
                sg = s_v[g, h_off, pl.ds(woff, _LANES)]
                for s in range(_SAMPLES):
                    x = cg + jnp.float32(s - _SAMPLES // 2) * sg
                    xt = x.astype(jnp.int32)
                    f0 = jnp.where(x < xt.astype(jnp.float32), xt - 1, xt)
                    for l in levels:
                        wl = level_w[l]
                        xi = x * jnp.float32(1.0 / (2 ** l)) if l else x
                        f = lax.shift_right_arithmetic(f0, l) if l else f0
                        w1 = xi - f.astype(jnp.float32)
                        i1 = f + 1
                        c0 = jnp.clip(f, 0, wl - 1)
                        c1 = jnp.clip(i1, 0, wl - 1)
                        if l == 3:
                            v0 = plsc.load_gather(bufs[p][3], [lane, c0 + 64])
                            v1 = plsc.load_gather(bufs[p][3], [lane, c1 + 64])
                        else:
                            v0 = plsc.load_gather(bufs[p][l], [lane, c0])
                            v1 = plsc.load_gather(bufs[p][l], [lane, c1])
                        v0 = jnp.where(f == c0, v0, jnp.float32(0.0))
                        v1 = jnp.where(i1 == c1, v1, jnp.float32(0.0))
                        ch = l * _GS + g * _SAMPLES + s - half * hcw
                        o_v[ch, h_off, pl.ds(woff, _LANES)] = (
                            v0 + w1 * (v1 - v0))

        def run_pass(half, first):
            def body(k2, carry):
                for qp in range(2):
                    k = k2 * 2 + qp
                    wait_in(k, qp, half)
                    if qp == 1:
                        @pl.when(k2 < 31)
                        def _start_next():
                            start_in(k + 1, 0, half)
                    else:
                        start_in(k + 1, 1, half)
                    compute(k, qp, half)
                return carry

            if not first:
                out_copy(0).wait()
            lax.fori_loop(0, 32, body, 0)
            out_copy(half).start()

        for cp in coord_copies(sem_o):
            cp.start()
        start_in(0, 0, 0)
        for cp in coord_copies(sem_o):
            cp.wait()
        run_pass(0, True)
        start_in(0, 0, 1)
        run_pass(1, False)
        out_copy(1).wait()

    return sampler


def kernel(fmap1, fmap2, coords, sigma):
    b, c, h, w = fmap1.shape
    o0, o1, o23 = _corr_pyramid(fmap1, fmap2)
    sampler = _make_sc_sampler(b, h, w)
    return sampler(o0, o1, o23, coords, sigma)


# speedup vs baseline: 1.0649x; 1.0057x over previous
import functools
import math

import jax
import jax.numpy as jnp
from jax import lax
from jax.experimental import pallas as pl
from jax.experimental.pallas import tpu as pltpu
from jax.experimental.pallas import tpu_sc as plsc

_SAMPLES = 9
_NLVL = 4
_G = 4
_GS = _G * _SAMPLES
_CH = _NLVL * _GS
_LANES = 16
_HB = 32


def _pool_matrix(wl, wn, scale):
    r = lax.broadcasted_iota(jnp.int32, (wl, wn), 0)
    c = lax.broadcasted_iota(jnp.int32, (wl, wn), 1)
    return jnp.where(r // 2 == c, jnp.float32(scale), jnp.float32(0.0))


def _corr_pyr_body(f1_ref, f2_ref, o0_ref, o1_ref, o23_ref):
    c = f1_ref.shape[1]
    scale = jnp.float32(1.0 / math.sqrt(float(c)))
    p1 = _pool_matrix(256, 128, 0.5)
    r = lax.broadcasted_iota(jnp.int32, (128, 128), 0)
    cc = lax.broadcasted_iota(jnp.int32, (128, 128), 1)
    q = jnp.where(
        (cc < 64) & (r // 2 == cc), jnp.float32(0.5),
        jnp.where((cc >= 64) & (cc < 96) & (r // 4 == cc - 64),
                  jnp.float32(0.25), jnp.float32(0.0)))
    for hh in range(_HB):
        f1 = f1_ref[0, :, hh, :]
        f2 = f2_ref[0, :, hh, :]
        corr = lax.dot_general(f1, f2, (((0,), (0,)), ((), ())),
                               preferred_element_type=jnp.float32) * scale
        o0_ref[0, hh] = jnp.concatenate([corr[:, :128], corr[:, 128:]], axis=0)
        l1 = lax.dot_general(corr, p1, (((1,), (0,)), ((), ())),
                             preferred_element_type=jnp.float32)
        o1_ref[0, hh] = l1
        o23_ref[0, hh] = lax.dot_general(l1, q, (((1,), (0,)), ((), ())),
                                         preferred_element_type=jnp.float32)


def _corr_pyramid(f1, f2):
    b, c, h, w = f1.shape
    shapes = [(b, h, 2 * w, 128), (b, h, w, 128), (b, h, w, 128)]
    return pl.pallas_call(
        _corr_pyr_body,
        grid=(b, h // _HB),
        in_specs=[pl.BlockSpec((1, c, _HB, w), lambda i, j: (i, 0, j, 0))] * 2,
        out_specs=[pl.BlockSpec((1, _HB, s[2], 128), lambda i, j: (i, j, 0, 0))
                   for s in shapes],
        out_shape=[jax.ShapeDtypeStruct(s, jnp.float32) for s in shapes],
    )(f1, f2)


def _make_sc_sampler(b_sz, h_sz, w):
    level_w = [w // (2 ** l) for l in range(_NLVL)]
    n_chunks = b_sz * h_sz * (w // _LANES)
    mesh = plsc.VectorSubcoreMesh(core_axis_name="c", subcore_axis_name="s")
    wc = w // _LANES
    hcw = _CH // 2

    @functools.partial(
        pl.kernel,
        mesh=mesh,
        compiler_params=pltpu.CompilerParams(needs_layout_passes=False),
        out_type=jax.ShapeDtypeStruct((b_sz, _CH, h_sz, w), jnp.float32),
        scratch_types=(
            [pltpu.VMEM((_LANES, 256), jnp.float32) for _ in range(2)]
            + [pltpu.VMEM((_LANES, 128), jnp.float32) for _ in range(4)]
            + [pltpu.VMEM((_G, 8, 128), jnp.float32) for _ in range(2)]
            + [pltpu.VMEM((hcw, 8, 128), jnp.float32)]
            + [pltpu.SemaphoreType.DMA for _ in range(3)]
        ),
    )
    def sampler(p0_hbm, p1_hbm, p23_hbm, crd_hbm, sig_hbm, out_hbm,
                p0a, p0b, p1a, p1b, p23a, p23b,
                c_v, s_v, o_v, sem_a, sem_b, sem_o):
        wid = lax.axis_index("s") * 2 + lax.axis_index("c")
        bi = lax.shift_right_logical(wid, 4)
        hb = lax.shift_right_logical(wid, 1) & 7
        wq = wid & 1
        lane = lax.broadcasted_iota(jnp.int32, (_LANES,), 0)
        bufs = [[p0a, p1a, p23a, p23a], [p0b, p1b, p23b, p23b]]
        sem_in = [sem_a, sem_b]

        def chunk_coords(k):
            hi = hb * 8 + lax.shift_right_logical(k, 3)
            s = wq * 8 + (k & 7)
            t = (bi * h_sz + hi) * wc + s
            return hi, s, t

        def in_copies(k, p, half):
            hi, s, t = chunk_coords(k)
            r0 = s * _LANES
            if half == 0:
                p0_v = bufs[p][0]
                yield pltpu.make_async_copy(
                    p0_hbm.at[bi, hi, pl.ds(r0, _LANES)],
                    p0_v.at[:, pl.ds(0, 128)], sem_in[p])
                yield pltpu.make_async_copy(
                    p0_hbm.at[bi, hi, pl.ds(w + r0, _LANES)],
                    p0_v.at[:, pl.ds(128, 128)], sem_in[p])
                yield pltpu.make_async_copy(
                    p1_hbm.at[bi, hi, pl.ds(r0, _LANES)], bufs[p][1],
                    sem_in[p])
            else:
                yield pltpu.make_async_copy(
                    p23_hbm.at[bi, hi, pl.ds(r0, _LANES)], bufs[p][2],
                    sem_in[p])

        def coord_copies(sem):
            yield pltpu.make_async_copy(
                crd_hbm.at[bi, :, pl.ds(hb * 8, 8), pl.ds(wq * 128, 128)],
                c_v, sem)
            yield pltpu.make_async_copy(
                sig_hbm.at[bi, :, pl.ds(hb * 8, 8), pl.ds(wq * 128, 128)],
                s_v, sem)

        def start_in(k, p, half):
            for cp in in_copies(k, p, half):
                cp.start()

        def wait_in(k, p, half):
            for cp in in_copies(k, p, half):
                cp.wait()

        def out_copy(half):
            return pltpu.make_async_copy(
                o_v,
                out_hbm.at[bi, pl.ds(half * hcw, hcw),
                           pl.ds(hb * 8, 8), pl.ds(wq * 128, 128)],
                sem_o)

        def compute(k, p, half):
            h_off = lax.shift_right_logical(k, 3)
            woff = (k & 7) * _LANES
            levels = (0, 1) if half == 0 else (2, 3)
            for g in range(_G):
                cg = c_v[g, h_off, pl.ds(woff, _LANES)]
                sg = s_v[g, h_off, pl.ds(woff, _LANES)]
                for s in range(_SAMPLES):
                    x = cg + jnp.float32(s - _SAMPLES // 2) * sg
                    xt = x.astype(jnp.int32)
                    f0 = jnp.where(x < xt.astype(jnp.float32), xt - 1, xt)
                    for l in levels:
                        wl = level_w[l]
                        xi = x * jnp.float32(1.0 / (2 ** l)) if l else x
                        f = lax.shift_right_arithmetic(f0, l) if l else f0
                        w1 = xi - f.astype(jnp.float32)
                        i1 = f + 1
                        c0 = jnp.clip(f, 0, wl - 1)
                        c1 = jnp.clip(i1, 0, wl - 1)
                        if l == 3:
                            v0 = plsc.load_gather(bufs[p][3], [lane, c0 + 64])
                            v1 = plsc.load_gather(bufs[p][3], [lane, c1 + 64])
                        else:
                            v0 = plsc.load_gather(bufs[p][l], [lane, c0])
                            v1 = plsc.load_gather(bufs[p][l], [lane, c1])
                        v0 = jnp.where(f == c0, v0, jnp.float32(0.0))
                        v1 = jnp.where(i1 == c1, v1, jnp.float32(0.0))
                        ch = l * _GS + g * _SAMPLES + s - half * hcw
                        o_v[ch, h_off, pl.ds(woff, _LANES)] = (
                            v0 + w1 * (v1 - v0))

        def run_pass(half, first):
            def body(k2, carry):
                for qp in range(2):
                    k = k2 * 2 + qp
                    wait_in(k, qp, half)
                    if qp == 1:
                        @pl.when(k2 < 31)
                        def _start_next():
                            start_in(k + 1, 0, half)
                    else:
                        start_in(k + 1, 1, half)
                    compute(k, qp, half)
                return carry

            if not first:
                out_copy(0).wait()
            lax.fori_loop(0, 32, body, 0)
            out_copy(half).start()

        for cp in coord_copies(sem_o):
            cp.start()
        start_in(0, 0, 0)
        for cp in coord_copies(sem_o):
            cp.wait()
        run_pass(0, True)
        start_in(0, 0, 1)
        run_pass(1, False)
        out_copy(1).wait()

    return sampler


def kernel(fmap1, fmap2, coords, sigma):
    b, c, h, w = fmap1.shape
    o0, o1, o23 = _corr_pyramid(fmap1, fmap2)
    sampler = _make_sc_sampler(b, h, w)
    return sampler(o0, o1, o23, coords, sigma)
